# baseline (speedup 1.0000x reference)
import functools

import jax
import jax.numpy as jnp
from jax import lax
from jax.experimental import pallas as pl
from jax.experimental.pallas import tpu as pltpu
from jax.experimental.pallas import tpu_sc as plsc

N_NODES = 10000
N_EDGES = 320000
D = 128
L = 16
NC = 2
NS = 16
NW = NC * NS
NHALF = N_NODES // NC
EDGES_PER_W = N_EDGES // NW
BLK = 2000
NBLK = EDGES_PER_W // BLK
CHUNK = 16
PAIR_SHIFT = 5
CAP = 10496
ZROWS = 312
TAIL = NHALF - NS * ZROWS

_mesh = plsc.VectorSubcoreMesh(core_axis_name="c", subcore_axis_name="s")


@functools.partial(
    pl.kernel,
    out_type=(
        jax.ShapeDtypeStruct((NC * NW * CAP,), jnp.int32),
        jax.ShapeDtypeStruct((NC * NW * CAP,), jnp.int32),
        jax.ShapeDtypeStruct((NC * NW * CAP,), jnp.float32),
        jax.ShapeDtypeStruct((NW * L,), jnp.int32),
    ),
    mesh=_mesh,
    scratch_types=[
        pltpu.VMEM((BLK,), jnp.int32),
        pltpu.VMEM((BLK,), jnp.int32),
        pltpu.VMEM((BLK,), jnp.int32),
        pltpu.VMEM((BLK,), jnp.int32),
        pltpu.VMEM((BLK,), jnp.float32),
        pltpu.VMEM((BLK,), jnp.float32),
        pltpu.VMEM((CAP,), jnp.int32),
        pltpu.VMEM((CAP,), jnp.int32),
        pltpu.VMEM((CAP,), jnp.float32),
        pltpu.VMEM((CAP,), jnp.int32),
        pltpu.VMEM((CAP,), jnp.int32),
        pltpu.VMEM((CAP,), jnp.float32),
        pltpu.VMEM((L,), jnp.int32),
        pltpu.SemaphoreType.DMA,
        pltpu.SemaphoreType.DMA,
        pltpu.SemaphoreType.DMA,
    ],
    compiler_params=pltpu.CompilerParams(needs_layout_passes=False),
)
def _route_sc(src_hbm, dst_hbm, w_hbm,
              srcR, dstR, wR, cntR,
              rsrc0, rsrc1, rdst0, rdst1, rw0, rw1,
              ls0, ld0, lw0, ls1, ld1, lw1, cntv,
              isem0, isem1, osem):
    c = lax.axis_index("c")
    s = lax.axis_index("s")
    wid = s * NC + c
    ebase = wid * EDGES_PER_W
    rsrc = (rsrc0, rsrc1)
    rdst = (rdst0, rdst1)
    rw = (rw0, rw1)
    isem = (isem0, isem1)

    def _issue_blk(k, b):
        base = ebase + k * BLK
        pltpu.async_copy(src_hbm.at[pl.ds(base, BLK)], rsrc[b], isem[b])
        pltpu.async_copy(dst_hbm.at[pl.ds(base, BLK)], rdst[b], isem[b])
        pltpu.async_copy(w_hbm.at[pl.ds(base, BLK)], rw[b], isem[b])

    def _wait_blk(b):
        pltpu.make_async_copy(src_hbm.at[pl.ds(0, BLK)], rsrc[b],
                              isem[b]).wait()
        pltpu.make_async_copy(dst_hbm.at[pl.ds(0, BLK)], rdst[b],
                              isem[b]).wait()
        pltpu.make_async_copy(w_hbm.at[pl.ds(0, BLK)], rw[b],
                              isem[b]).wait()

    _issue_blk(0, 0)
    if NBLK > 1:
        _issue_blk(1, 1)

    cnt0 = jnp.int32(0)
    cnt1 = jnp.int32(0)
    for k in range(NBLK):
        b = k % 2
        _wait_blk(b)

        def _step(i, carry, b=b):
            c0, c1 = carry
            sl = pl.ds(i * L, L)
            sv = rsrc[b][sl]
            dv = rdst[b][sl]
            wv = rw[b][sl]
            m0 = dv < NHALF
            plsc.store_compressed(ls0.at[pl.ds(c0, L)], sv, mask=m0)
            plsc.store_compressed(ld0.at[pl.ds(c0, L)], dv, mask=m0)
            plsc.store_compressed(lw0.at[pl.ds(c0, L)], wv, mask=m0)
            m1 = ~m0
            plsc.store_compressed(ls1.at[pl.ds(c1, L)], sv, mask=m1)
            plsc.store_compressed(ld1.at[pl.ds(c1, L)], dv - NHALF, mask=m1)
            plsc.store_compressed(lw1.at[pl.ds(c1, L)], wv, mask=m1)
            n0 = lax.reduce_max(plsc.all_reduce_population_count(m0), (0,))
            return c0 + n0, c1 + (L - n0)

        cnt0, cnt1 = lax.fori_loop(0, BLK // L, _step, (cnt0, cnt1))
        if k + 2 < NBLK:
            _issue_blk(k + 2, b)

    zero16i = jnp.zeros((L,), jnp.int32)
    zero16f = jnp.zeros((L,), jnp.float32)

    def _pad(cnt, lsb, ldb, lwb):
        npairs = lax.max((cnt + 2 * CHUNK - 1) >> PAIR_SHIFT, 1)
        padded = npairs << PAIR_SHIFT

        def _padv(k, _):
            off = cnt + k * L
            lsb[pl.ds(off, L)] = zero16i
            ldb[pl.ds(off, L)] = zero16i
            lwb[pl.ds(off, L)] = zero16f
            return 0

        lax.fori_loop(0, (padded - cnt + L - 1) >> 4, _padv, 0)
        return npairs

    np0 = _pad(cnt0, ls0, ld0, lw0)
    np1 = _pad(cnt1, ls1, ld1, lw1)

    iota = lax.iota(jnp.int32, L)
    cntv[...] = jnp.where(iota == 0, np0, jnp.where(iota == 1, np1, 0))

    slot0 = (0 * NW + wid) * CAP
    slot1 = (1 * NW + wid) * CAP
    pltpu.async_copy(ls0, srcR.at[pl.ds(slot0, CAP)], osem)
    pltpu.async_copy(ld0, dstR.at[pl.ds(slot0, CAP)], osem)
    pltpu.async_copy(lw0, wR.at[pl.ds(slot0, CAP)], osem)
    pltpu.async_copy(ls1, srcR.at[pl.ds(slot1, CAP)], osem)
    pltpu.async_copy(ld1, dstR.at[pl.ds(slot1, CAP)], osem)
    pltpu.async_copy(lw1, wR.at[pl.ds(slot1, CAP)], osem)
    pltpu.async_copy(cntv, cntR.at[pl.ds(wid * L, L)], osem)
    pltpu.make_async_copy(ls0, srcR.at[pl.ds(slot0, CAP)], osem).wait()
    pltpu.make_async_copy(ld0, dstR.at[pl.ds(slot0, CAP)], osem).wait()
    pltpu.make_async_copy(lw0, wR.at[pl.ds(slot0, CAP)], osem).wait()
    pltpu.make_async_copy(ls1, srcR.at[pl.ds(slot1, CAP)], osem).wait()
    pltpu.make_async_copy(ld1, dstR.at[pl.ds(slot1, CAP)], osem).wait()
    pltpu.make_async_copy(lw1, wR.at[pl.ds(slot1, CAP)], osem).wait()
    pltpu.make_async_copy(cntv, cntR.at[pl.ds(wid * L, L)], osem).wait()


@functools.partial(
    pl.kernel,
    out_type=jax.ShapeDtypeStruct((NC, NHALF, D), jnp.float32),
    mesh=_mesh,
    scratch_types=[
        pltpu.VMEM((2 * CHUNK,), jnp.int32),
        pltpu.VMEM((2 * CHUNK,), jnp.int32),
        pltpu.VMEM((2 * CHUNK,), jnp.int32),
        pltpu.VMEM((2 * CHUNK,), jnp.int32),
        pltpu.VMEM((2 * CHUNK,), jnp.float32),
        pltpu.VMEM((2 * CHUNK,), jnp.float32),
        pltpu.VMEM((CHUNK,), jnp.int32),
        pltpu.VMEM((CHUNK,), jnp.int32),
        pltpu.VMEM((CHUNK,), jnp.int32),
        pltpu.VMEM((CHUNK,), jnp.int32),
        pltpu.VMEM((CHUNK,), jnp.int32),
        pltpu.VMEM((CHUNK,), jnp.int32),
        pltpu.VMEM((CHUNK,), jnp.int32),
        pltpu.VMEM((CHUNK,), jnp.int32),
        pltpu.VMEM((CHUNK,), jnp.float32),
        pltpu.VMEM((CHUNK,), jnp.float32),
        pltpu.VMEM((CHUNK,), jnp.float32),
        pltpu.VMEM((CHUNK,), jnp.float32),
        pltpu.VMEM((CHUNK, D), jnp.float32),
        pltpu.VMEM((CHUNK, D), jnp.float32),
        pltpu.VMEM((CHUNK, D), jnp.float32),
        pltpu.VMEM((CHUNK, D), jnp.float32),
        pltpu.VMEM((L,), jnp.int32),
        pltpu.VMEM((ZROWS, D), jnp.float32),
        pltpu.VMEM((TAIL, D), jnp.float32),
        pltpu.VMEM_SHARED((NHALF, D), jnp.float32),
        pltpu.SemaphoreType.DMA,
        pltpu.SemaphoreType.DMA,
        pltpu.SemaphoreType.DMA,
        pltpu.SemaphoreType.DMA,
        pltpu.SemaphoreType.DMA,
        pltpu.SemaphoreType.DMA,
        pltpu.SemaphoreType.DMA,
        pltpu.SemaphoreType.DMA,
        pltpu.SemaphoreType.DMA,
        pltpu.SemaphoreType.DMA,
    ],
    compiler_params=pltpu.CompilerParams(needs_layout_passes=False),
)
def _propagate_sc(x_hbm, srcR, dstR, wR, cntR, out_hbm,
                  rsrc0, rsrc1, rdst0, rdst1, rw0, rw1,
                  srcv0, srcv1, srcv2, srcv3,
                  dstv0, dstv1, dstv2, dstv3,
                  wv0, wv1, wv2, wv3,
                  rows0, rows1, rows2, rows3, cntb, stage, stail, acc,
                  esem0, esem1, gsem0, gsem1, gsem2, gsem3,
                  ssem0, ssem1, ssem2, ssem3):
    rsrc = (rsrc0, rsrc1)
    rdst = (rdst0, rdst1)
    rw = (rw0, rw1)
    srcv = (srcv0, srcv1, srcv2, srcv3)
    dstv = (dstv0, dstv1, dstv2, dstv3)
    wv = (wv0, wv1, wv2, wv3)
    rows = (rows0, rows1, rows2, rows3)
    esem = (esem0, esem1)
    gsem = (gsem0, gsem1, gsem2, gsem3)
    ssem = (ssem0, ssem1, ssem2, ssem3)
    c = lax.axis_index("c")
    s = lax.axis_index("s")

    zero = jnp.zeros((L,), jnp.float32)

    def _zrow(i, _):
        for j in range(D // L):
            stage[i, pl.ds(j * L, L)] = zero
        return 0

    lax.fori_loop(0, ZROWS, _zrow, 0)
    for i in range(TAIL):
        for j in range(D // L):
            stail[i, pl.ds(j * L, L)] = zero
    pltpu.sync_copy(stage, acc.at[pl.ds(s * ZROWS, ZROWS)])

    @pl.when(s == NS - 1)
    def _():
        pltpu.sync_copy(stail, acc.at[pl.ds(NS * ZROWS, TAIL)])

    plsc.subcore_barrier()

    iota = lax.iota(jnp.int32, L)

    def _issue_e(base0, j, b):
        base = base0 + j * 2 * CHUNK
        pltpu.async_copy(srcR.at[pl.ds(base, 2 * CHUNK)], rsrc[b], esem[b])
        pltpu.async_copy(dstR.at[pl.ds(base, 2 * CHUNK)], rdst[b], esem[b])
        pltpu.async_copy(wR.at[pl.ds(base, 2 * CHUNK)], rw[b], esem[b])

    def _wait_e(b):
        pltpu.make_async_copy(srcR.at[pl.ds(0, 2 * CHUNK)], rsrc[b],
                              esem[b]).wait()
        pltpu.make_async_copy(dstR.at[pl.ds(0, 2 * CHUNK)], rdst[b],
                              esem[b]).wait()
        pltpu.make_async_copy(wR.at[pl.ds(0, 2 * CHUNK)], rw[b],
                              esem[b]).wait()

    def _build(eb, half, b):
        for k in range(CHUNK // L):
            sl = pl.ds(k * L, L)
            el = pl.ds(half * CHUNK + k * L, L)
            srcv[b][sl] = rsrc[eb][el]
            dstv[b][sl] = rdst[eb][el]
            wv[b][sl] = rw[eb][el]

    def _issue_g(b):
        pltpu.async_copy(x_hbm.at[srcv[b]], rows[b], gsem[b])

    def _wait_g(b):
        pltpu.make_async_copy(x_hbm.at[pl.ds(0, CHUNK)], rows[b],
                              gsem[b]).wait()

    def _scale(b):
        @plsc.parallel_loop(0, CHUNK, unroll=2)
        def _edge(e):
            wb = plsc.load_gather(wv[b], [jnp.full((L,), e, jnp.int32)])
            for j in range(D // L):
                sl = pl.ds(j * L, L)
                rows[b][e, sl] = rows[b][e, sl] * wb

    def _issue_a(b):
        pltpu.async_copy(rows[b], acc.at[dstv[b]], ssem[b], add=True)

    def _wait_a(b):
        pltpu.make_async_copy(x_hbm.at[pl.ds(0, CHUNK)], rows[b],
                              ssem[b]).wait()

    def _run_slot(w):
        pltpu.sync_copy(cntR.at[pl.ds(w * L, L)], cntb)
        npair = lax.reduce_max(jnp.where(iota == c, cntb[...], 0), (0,))
        base0 = (c * NW + w) * CAP

        nch = npair * 2

        _issue_e(base0, 0, 0)
        _issue_e(base0, 1, 1)
        _wait_e(0)
        _build(0, 0, 0)
        _issue_g(0)
        _build(0, 1, 1)
        _issue_g(1)

        @pl.when(2 < npair)
        def _():
            _issue_e(base0, 2, 0)

        def _quad(i, _):
            for k in range(4):
                ck = 4 * i + k
                p = k
                np_ = (k + 2) % 4
                ebb = 1 if k < 2 else 0
                half = k % 2

                @pl.when(ck < nch)
                def _(ck=ck, p=p, np_=np_, ebb=ebb, half=half, k=k):
                    @pl.when(ck + 2 < nch)
                    def _():
                        if k % 2 == 0:
                            _wait_e(ebb)
                        if k >= 2:
                            _wait_a(np_)
                        else:
                            @pl.when(ck >= 2)
                            def _():
                                _wait_a(np_)
                        _build(ebb, half, np_)
                        _issue_g(np_)
                        if k % 2 == 1:
                            jj = (ck + 2) // 2 + 2

                            @pl.when(jj < npair)
                            def _():
                                _issue_e(base0, jj, ebb)

                    _wait_g(p)
                    _scale(p)
                    _issue_a(p)
            return 0

        lax.fori_loop(0, (nch + 3) >> 2, _quad, 0)

        @pl.when(npair == 1)
        def _():
            _wait_e(1)
        _wait_a(0)
        _wait_a(1)

        @pl.when(npair >= 2)
        def _():
            _wait_a(2)
            _wait_a(3)

    _run_slot(s * 2)
    _run_slot(s * 2 + 1)
    plsc.subcore_barrier()

    r0 = s * ZROWS
    pltpu.sync_copy(acc.at[pl.ds(r0, ZROWS)], stage)
    pltpu.sync_copy(stage, out_hbm.at[c, pl.ds(r0, ZROWS)])

    @pl.when(s == NS - 1)
    def _():
        pltpu.sync_copy(acc.at[pl.ds(NS * ZROWS, TAIL)], stail)
        pltpu.sync_copy(stail, out_hbm.at[c, pl.ds(NS * ZROWS, TAIL)])


def _dense_relu_body(p_ref, wt_ref, b_ref, out_ref):
    y = jnp.dot(p_ref[...], wt_ref[...],
                preferred_element_type=jnp.float32) + b_ref[...]
    out_ref[...] = jnp.maximum(y, 0.0)


def _dense_final_body(p_ref, wt_ref, b_ref, out_ref):
    y = jnp.dot(p_ref[...], wt_ref[...],
                preferred_element_type=jnp.float32) + b_ref[...]
    m = jnp.max(y, axis=1, keepdims=True)
    e = y - m
    lse = jnp.log(jnp.sum(jnp.exp(e), axis=1, keepdims=True))
    out_ref[...] = e - lse


_ROW_BLK = 2000


def _dense_call(body, p, wt, b):
    grid = (N_NODES // _ROW_BLK,)
    return pl.pallas_call(
        body,
        grid=grid,
        in_specs=[
            pl.BlockSpec((_ROW_BLK, D), lambda i: (i, 0)),
            pl.BlockSpec((D, D), lambda i: (0, 0)),
            pl.BlockSpec((1, D), lambda i: (0, 0)),
        ],
        out_specs=pl.BlockSpec((_ROW_BLK, D), lambda i: (i, 0)),
        out_shape=jax.ShapeDtypeStruct((N_NODES, D), jnp.float32),
    )(p, wt, b)


def kernel(x, edge_index, edge_weight, W1, b1, W2, b2, W3, b3):
    src = edge_index[0].astype(jnp.int32)
    dst = edge_index[1].astype(jnp.int32)
    w = edge_weight.astype(jnp.float32)

    srcR, dstR, wR, cntR = _route_sc(src, dst, w)

    p1 = _propagate_sc(x, srcR, dstR, wR, cntR).reshape(N_NODES, D)
    h1 = _dense_call(_dense_relu_body, p1, W1.T, b1.reshape(1, D))
    p2 = _propagate_sc(h1, srcR, dstR, wR, cntR).reshape(N_NODES, D)
    h2 = _dense_call(_dense_relu_body, p2, W2.T, b2.reshape(1, D))
    p3 = _propagate_sc(h2, srcR, dstR, wR, cntR).reshape(N_NODES, D)
    return _dense_call(_dense_final_body, p3, W3.T, b3.reshape(1, D))


# speedup vs baseline: 1.0490x; 1.0490x over previous
import functools

import jax
import jax.numpy as jnp
from jax import lax
from jax.experimental import pallas as pl
from jax.experimental.pallas import tpu as pltpu
from jax.experimental.pallas import tpu_sc as plsc

N_NODES = 10000
N_EDGES = 320000
D = 128
L = 16
NC = 2
NS = 16
NW = NC * NS
NHALF = N_NODES // NC
EDGES_PER_W = N_EDGES // NW
BLK = 2000
NBLK = EDGES_PER_W // BLK
CHUNK = 32
PAIR_SHIFT = 6
CAP = 10496
ZROWS = 312
TAIL = NHALF - NS * ZROWS

_mesh = plsc.VectorSubcoreMesh(core_axis_name="c", subcore_axis_name="s")


@functools.partial(
    pl.kernel,
    out_type=(
        jax.ShapeDtypeStruct((NC * NW * CAP,), jnp.int32),
        jax.ShapeDtypeStruct((NC * NW * CAP,), jnp.int32),
        jax.ShapeDtypeStruct((NC * NW * CAP,), jnp.float32),
        jax.ShapeDtypeStruct((NW * L,), jnp.int32),
    ),
    mesh=_mesh,
    scratch_types=[
        pltpu.VMEM((BLK,), jnp.int32),
        pltpu.VMEM((BLK,), jnp.int32),
        pltpu.VMEM((BLK,), jnp.int32),
        pltpu.VMEM((BLK,), jnp.int32),
        pltpu.VMEM((BLK,), jnp.float32),
        pltpu.VMEM((BLK,), jnp.float32),
        pltpu.VMEM((CAP,), jnp.int32),
        pltpu.VMEM((CAP,), jnp.int32),
        pltpu.VMEM((CAP,), jnp.float32),
        pltpu.VMEM((CAP,), jnp.int32),
        pltpu.VMEM((CAP,), jnp.int32),
        pltpu.VMEM((CAP,), jnp.float32),
        pltpu.VMEM((L,), jnp.int32),
        pltpu.SemaphoreType.DMA,
        pltpu.SemaphoreType.DMA,
        pltpu.SemaphoreType.DMA,
    ],
    compiler_params=pltpu.CompilerParams(needs_layout_passes=False),
)
def _route_sc(src_hbm, dst_hbm, w_hbm,
              srcR, dstR, wR, cntR,
              rsrc0, rsrc1, rdst0, rdst1, rw0, rw1,
              ls0, ld0, lw0, ls1, ld1, lw1, cntv,
              isem0, isem1, osem):
    c = lax.axis_index("c")
    s = lax.axis_index("s")
    wid = s * NC + c
    ebase = wid * EDGES_PER_W
    rsrc = (rsrc0, rsrc1)
    rdst = (rdst0, rdst1)
    rw = (rw0, rw1)
    isem = (isem0, isem1)

    def _issue_blk(k, b):
        base = ebase + k * BLK
        pltpu.async_copy(src_hbm.at[pl.ds(base, BLK)], rsrc[b], isem[b])
        pltpu.async_copy(dst_hbm.at[pl.ds(base, BLK)], rdst[b], isem[b])
        pltpu.async_copy(w_hbm.at[pl.ds(base, BLK)], rw[b], isem[b])

    def _wait_blk(b):
        pltpu.make_async_copy(src_hbm.at[pl.ds(0, BLK)], rsrc[b],
                              isem[b]).wait()
        pltpu.make_async_copy(dst_hbm.at[pl.ds(0, BLK)], rdst[b],
                              isem[b]).wait()
        pltpu.make_async_copy(w_hbm.at[pl.ds(0, BLK)], rw[b],
                              isem[b]).wait()

    _issue_blk(0, 0)
    if NBLK > 1:
        _issue_blk(1, 1)

    cnt0 = jnp.int32(0)
    cnt1 = jnp.int32(0)
    for k in range(NBLK):
        b = k % 2
        _wait_blk(b)

        def _step(i, carry, b=b):
            c0, c1 = carry
            sl = pl.ds(i * L, L)
            sv = rsrc[b][sl]
            dv = rdst[b][sl]
            wv = rw[b][sl]
            m0 = dv < NHALF
            plsc.store_compressed(ls0.at[pl.ds(c0, L)], sv, mask=m0)
            plsc.store_compressed(ld0.at[pl.ds(c0, L)], dv, mask=m0)
            plsc.store_compressed(lw0.at[pl.ds(c0, L)], wv, mask=m0)
            m1 = ~m0
            plsc.store_compressed(ls1.at[pl.ds(c1, L)], sv, mask=m1)
            plsc.store_compressed(ld1.at[pl.ds(c1, L)], dv - NHALF, mask=m1)
            plsc.store_compressed(lw1.at[pl.ds(c1, L)], wv, mask=m1)
            n0 = lax.reduce_max(plsc.all_reduce_population_count(m0), (0,))
            return c0 + n0, c1 + (L - n0)

        cnt0, cnt1 = lax.fori_loop(0, BLK // L, _step, (cnt0, cnt1))
        if k + 2 < NBLK:
            _issue_blk(k + 2, b)

    zero16i = jnp.zeros((L,), jnp.int32)
    zero16f = jnp.zeros((L,), jnp.float32)

    def _pad(cnt, lsb, ldb, lwb):
        npairs = lax.max((cnt + 2 * CHUNK - 1) >> PAIR_SHIFT, 1)
        padded = npairs << PAIR_SHIFT

        def _padv(k, _):
            off = cnt + k * L
            lsb[pl.ds(off, L)] = zero16i
            ldb[pl.ds(off, L)] = zero16i
            lwb[pl.ds(off, L)] = zero16f
            return 0

        lax.fori_loop(0, (padded - cnt + L - 1) >> 4, _padv, 0)
        return npairs

    np0 = _pad(cnt0, ls0, ld0, lw0)
    np1 = _pad(cnt1, ls1, ld1, lw1)

    iota = lax.iota(jnp.int32, L)
    cntv[...] = jnp.where(iota == 0, np0, jnp.where(iota == 1, np1, 0))

    slot0 = (0 * NW + wid) * CAP
    slot1 = (1 * NW + wid) * CAP
    pltpu.async_copy(ls0, srcR.at[pl.ds(slot0, CAP)], osem)
    pltpu.async_copy(ld0, dstR.at[pl.ds(slot0, CAP)], osem)
    pltpu.async_copy(lw0, wR.at[pl.ds(slot0, CAP)], osem)
    pltpu.async_copy(ls1, srcR.at[pl.ds(slot1, CAP)], osem)
    pltpu.async_copy(ld1, dstR.at[pl.ds(slot1, CAP)], osem)
    pltpu.async_copy(lw1, wR.at[pl.ds(slot1, CAP)], osem)
    pltpu.async_copy(cntv, cntR.at[pl.ds(wid * L, L)], osem)
    pltpu.make_async_copy(ls0, srcR.at[pl.ds(slot0, CAP)], osem).wait()
    pltpu.make_async_copy(ld0, dstR.at[pl.ds(slot0, CAP)], osem).wait()
    pltpu.make_async_copy(lw0, wR.at[pl.ds(slot0, CAP)], osem).wait()
    pltpu.make_async_copy(ls1, srcR.at[pl.ds(slot1, CAP)], osem).wait()
    pltpu.make_async_copy(ld1, dstR.at[pl.ds(slot1, CAP)], osem).wait()
    pltpu.make_async_copy(lw1, wR.at[pl.ds(slot1, CAP)], osem).wait()
    pltpu.make_async_copy(cntv, cntR.at[pl.ds(wid * L, L)], osem).wait()


@functools.partial(
    pl.kernel,
    out_type=jax.ShapeDtypeStruct((NC, NHALF, D), jnp.float32),
    mesh=_mesh,
    scratch_types=[
        pltpu.VMEM((2 * CHUNK,), jnp.int32),
        pltpu.VMEM((2 * CHUNK,), jnp.int32),
        pltpu.VMEM((2 * CHUNK,), jnp.int32),
        pltpu.VMEM((2 * CHUNK,), jnp.int32),
        pltpu.VMEM((2 * CHUNK,), jnp.float32),
        pltpu.VMEM((2 * CHUNK,), jnp.float32),
        pltpu.VMEM((CHUNK,), jnp.int32),
        pltpu.VMEM((CHUNK,), jnp.int32),
        pltpu.VMEM((CHUNK,), jnp.int32),
        pltpu.VMEM((CHUNK,), jnp.int32),
        pltpu.VMEM((CHUNK,), jnp.int32),
        pltpu.VMEM((CHUNK,), jnp.int32),
        pltpu.VMEM((CHUNK,), jnp.int32),
        pltpu.VMEM((CHUNK,), jnp.int32),
        pltpu.VMEM((CHUNK,), jnp.float32),
        pltpu.VMEM((CHUNK,), jnp.float32),
        pltpu.VMEM((CHUNK,), jnp.float32),
        pltpu.VMEM((CHUNK,), jnp.float32),
        pltpu.VMEM((CHUNK, D), jnp.float32),
        pltpu.VMEM((CHUNK, D), jnp.float32),
        pltpu.VMEM((CHUNK, D), jnp.float32),
        pltpu.VMEM((CHUNK, D), jnp.float32),
        pltpu.VMEM((L,), jnp.int32),
        pltpu.VMEM((ZROWS, D), jnp.float32),
        pltpu.VMEM((TAIL, D), jnp.float32),
        pltpu.VMEM_SHARED((NHALF, D), jnp.float32),
        pltpu.SemaphoreType.DMA,
        pltpu.SemaphoreType.DMA,
        pltpu.SemaphoreType.DMA,
        pltpu.SemaphoreType.DMA,
        pltpu.SemaphoreType.DMA,
        pltpu.SemaphoreType.DMA,
        pltpu.SemaphoreType.DMA,
        pltpu.SemaphoreType.DMA,
        pltpu.SemaphoreType.DMA,
        pltpu.SemaphoreType.DMA,
    ],
    compiler_params=pltpu.CompilerParams(needs_layout_passes=False),
)
def _propagate_sc(x_hbm, srcR, dstR, wR, cntR, out_hbm,
                  rsrc0, rsrc1, rdst0, rdst1, rw0, rw1,
                  srcv0, srcv1, srcv2, srcv3,
                  dstv0, dstv1, dstv2, dstv3,
                  wv0, wv1, wv2, wv3,
                  rows0, rows1, rows2, rows3, cntb, stage, stail, acc,
                  esem0, esem1, gsem0, gsem1, gsem2, gsem3,
                  ssem0, ssem1, ssem2, ssem3):
    rsrc = (rsrc0, rsrc1)
    rdst = (rdst0, rdst1)
    rw = (rw0, rw1)
    srcv = (srcv0, srcv1, srcv2, srcv3)
    dstv = (dstv0, dstv1, dstv2, dstv3)
    wv = (wv0, wv1, wv2, wv3)
    rows = (rows0, rows1, rows2, rows3)
    esem = (esem0, esem1)
    gsem = (gsem0, gsem1, gsem2, gsem3)
    ssem = (ssem0, ssem1, ssem2, ssem3)
    c = lax.axis_index("c")
    s = lax.axis_index("s")

    zero = jnp.zeros((L,), jnp.float32)

    def _zrow(i, _):
        for j in range(D // L):
            stage[i, pl.ds(j * L, L)] = zero
        return 0

    lax.fori_loop(0, ZROWS, _zrow, 0)
    for i in range(TAIL):
        for j in range(D // L):
            stail[i, pl.ds(j * L, L)] = zero
    pltpu.sync_copy(stage, acc.at[pl.ds(s * ZROWS, ZROWS)])

    @pl.when(s == NS - 1)
    def _():
        pltpu.sync_copy(stail, acc.at[pl.ds(NS * ZROWS, TAIL)])

    plsc.subcore_barrier()

    iota = lax.iota(jnp.int32, L)

    def _issue_e(base0, j, b):
        base = base0 + j * 2 * CHUNK
        pltpu.async_copy(srcR.at[pl.ds(base, 2 * CHUNK)], rsrc[b], esem[b])
        pltpu.async_copy(dstR.at[pl.ds(base, 2 * CHUNK)], rdst[b], esem[b])
        pltpu.async_copy(wR.at[pl.ds(base, 2 * CHUNK)], rw[b], esem[b])

    def _wait_e(b):
        pltpu.make_async_copy(srcR.at[pl.ds(0, 2 * CHUNK)], rsrc[b],
                              esem[b]).wait()
        pltpu.make_async_copy(dstR.at[pl.ds(0, 2 * CHUNK)], rdst[b],
                              esem[b]).wait()
        pltpu.make_async_copy(wR.at[pl.ds(0, 2 * CHUNK)], rw[b],
                              esem[b]).wait()

    def _build(eb, half, b):
        for k in range(CHUNK // L):
            sl = pl.ds(k * L, L)
            el = pl.ds(half * CHUNK + k * L, L)
            srcv[b][sl] = rsrc[eb][el]
            dstv[b][sl] = rdst[eb][el]
            wv[b][sl] = rw[eb][el]

    def _issue_g(b):
        pltpu.async_copy(x_hbm.at[srcv[b]], rows[b], gsem[b])

    def _wait_g(b):
        pltpu.make_async_copy(x_hbm.at[pl.ds(0, CHUNK)], rows[b],
                              gsem[b]).wait()

    def _scale(b):
        @plsc.parallel_loop(0, CHUNK, unroll=2)
        def _edge(e):
            wb = plsc.load_gather(wv[b], [jnp.full((L,), e, jnp.int32)])
            for j in range(D // L):
                sl = pl.ds(j * L, L)
                rows[b][e, sl] = rows[b][e, sl] * wb

    def _issue_a(b):
        pltpu.async_copy(rows[b], acc.at[dstv[b]], ssem[b], add=True)

    def _wait_a(b):
        pltpu.make_async_copy(x_hbm.at[pl.ds(0, CHUNK)], rows[b],
                              ssem[b]).wait()

    def _run_slot(w):
        pltpu.sync_copy(cntR.at[pl.ds(w * L, L)], cntb)
        npair = lax.reduce_max(jnp.where(iota == c, cntb[...], 0), (0,))
        base0 = (c * NW + w) * CAP

        nch = npair * 2

        _issue_e(base0, 0, 0)
        _issue_e(base0, 1, 1)
        _wait_e(0)
        _build(0, 0, 0)
        _issue_g(0)
        _build(0, 1, 1)
        _issue_g(1)

        @pl.when(2 < npair)
        def _():
            _issue_e(base0, 2, 0)

        def _quad(i, _):
            for k in range(4):
                ck = 4 * i + k
                p = k
                np_ = (k + 2) % 4
                ebb = 1 if k < 2 else 0
                half = k % 2

                @pl.when(ck < nch)
                def _(ck=ck, p=p, np_=np_, ebb=ebb, half=half, k=k):
                    @pl.when(ck + 2 < nch)
                    def _():
                        if k % 2 == 0:
                            _wait_e(ebb)
                        if k >= 2:
                            _wait_a(np_)
                        else:
                            @pl.when(ck >= 2)
                            def _():
                                _wait_a(np_)
                        _build(ebb, half, np_)
                        _issue_g(np_)
                        if k % 2 == 1:
                            jj = (ck + 2) // 2 + 2

                            @pl.when(jj < npair)
                            def _():
                                _issue_e(base0, jj, ebb)

                    _wait_g(p)
                    _scale(p)
                    _issue_a(p)
            return 0

        lax.fori_loop(0, (nch + 3) >> 2, _quad, 0)

        @pl.when(npair == 1)
        def _():
            _wait_e(1)
        _wait_a(0)
        _wait_a(1)

        @pl.when(npair >= 2)
        def _():
            _wait_a(2)
            _wait_a(3)

    _run_slot(s * 2)
    _run_slot(s * 2 + 1)
    plsc.subcore_barrier()

    r0 = s * ZROWS
    pltpu.sync_copy(acc.at[pl.ds(r0, ZROWS)], stage)
    pltpu.sync_copy(stage, out_hbm.at[c, pl.ds(r0, ZROWS)])

    @pl.when(s == NS - 1)
    def _():
        pltpu.sync_copy(acc.at[pl.ds(NS * ZROWS, TAIL)], stail)
        pltpu.sync_copy(stail, out_hbm.at[c, pl.ds(NS * ZROWS, TAIL)])


def _dense_relu_body(p_ref, wt_ref, b_ref, out_ref):
    y = jnp.dot(p_ref[...], wt_ref[...],
                preferred_element_type=jnp.float32) + b_ref[...]
    out_ref[...] = jnp.maximum(y, 0.0)


def _dense_final_body(p_ref, wt_ref, b_ref, out_ref):
    y = jnp.dot(p_ref[...], wt_ref[...],
                preferred_element_type=jnp.float32) + b_ref[...]
    m = jnp.max(y, axis=1, keepdims=True)
    e = y - m
    lse = jnp.log(jnp.sum(jnp.exp(e), axis=1, keepdims=True))
    out_ref[...] = e - lse


_ROW_BLK = 2000


def _dense_call(body, p, wt, b):
    grid = (N_NODES // _ROW_BLK,)
    return pl.pallas_call(
        body,
        grid=grid,
        in_specs=[
            pl.BlockSpec((_ROW_BLK, D), lambda i: (i, 0)),
            pl.BlockSpec((D, D), lambda i: (0, 0)),
            pl.BlockSpec((1, D), lambda i: (0, 0)),
        ],
        out_specs=pl.BlockSpec((_ROW_BLK, D), lambda i: (i, 0)),
        out_shape=jax.ShapeDtypeStruct((N_NODES, D), jnp.float32),
    )(p, wt, b)


def kernel(x, edge_index, edge_weight, W1, b1, W2, b2, W3, b3):
    src = edge_index[0].astype(jnp.int32)
    dst = edge_index[1].astype(jnp.int32)
    w = edge_weight.astype(jnp.float32)

    srcR, dstR, wR, cntR = _route_sc(src, dst, w)

    p1 = _propagate_sc(x, srcR, dstR, wR, cntR).reshape(N_NODES, D)
    h1 = _dense_call(_dense_relu_body, p1, W1.T, b1.reshape(1, D))
    p2 = _propagate_sc(h1, srcR, dstR, wR, cntR).reshape(N_NODES, D)
    h2 = _dense_call(_dense_relu_body, p2, W2.T, b2.reshape(1, D))
    p3 = _propagate_sc(h2, srcR, dstR, wR, cntR).reshape(N_NODES, D)
    return _dense_call(_dense_final_body, p3, W3.T, b3.reshape(1, D))


# speedup vs baseline: 1.0506x; 1.0016x over previous
import functools

import jax
import jax.numpy as jnp
from jax import lax
from jax.experimental import pallas as pl
from jax.experimental.pallas import tpu as pltpu
from jax.experimental.pallas import tpu_sc as plsc

N_NODES = 10000
N_EDGES = 320000
D = 128
L = 16
NC = 2
NS = 16
NW = NC * NS
NHALF = N_NODES // NC
EDGES_PER_W = N_EDGES // NW
BLK = 2000
NBLK = EDGES_PER_W // BLK
CHUNK = 32
PAIR_SHIFT = 6
CAP = 10496
ZROWS = 312
TAIL = NHALF - NS * ZROWS

_mesh = plsc.VectorSubcoreMesh(core_axis_name="c", subcore_axis_name="s")


@functools.partial(
    pl.kernel,
    out_type=(
        jax.ShapeDtypeStruct((NC * NW * CAP,), jnp.int32),
        jax.ShapeDtypeStruct((NC * NW * CAP,), jnp.int32),
        jax.ShapeDtypeStruct((NC * NW * CAP,), jnp.float32),
        jax.ShapeDtypeStruct((NW * L,), jnp.int32),
    ),
    mesh=_mesh,
    scratch_types=[
        pltpu.VMEM((BLK,), jnp.int32),
        pltpu.VMEM((BLK,), jnp.int32),
        pltpu.VMEM((BLK,), jnp.int32),
        pltpu.VMEM((BLK,), jnp.int32),
        pltpu.VMEM((BLK,), jnp.float32),
        pltpu.VMEM((BLK,), jnp.float32),
        pltpu.VMEM((CAP,), jnp.int32),
        pltpu.VMEM((CAP,), jnp.int32),
        pltpu.VMEM((CAP,), jnp.float32),
        pltpu.VMEM((CAP,), jnp.int32),
        pltpu.VMEM((CAP,), jnp.int32),
        pltpu.VMEM((CAP,), jnp.float32),
        pltpu.VMEM((L,), jnp.int32),
        pltpu.SemaphoreType.DMA,
        pltpu.SemaphoreType.DMA,
        pltpu.SemaphoreType.DMA,
    ],
    compiler_params=pltpu.CompilerParams(needs_layout_passes=False),
)
def _route_sc(src_hbm, dst_hbm, w_hbm,
              srcR, dstR, wR, cntR,
              rsrc0, rsrc1, rdst0, rdst1, rw0, rw1,
              ls0, ld0, lw0, ls1, ld1, lw1, cntv,
              isem0, isem1, osem):
    c = lax.axis_index("c")
    s = lax.axis_index("s")
    wid = s * NC + c
    ebase = wid * EDGES_PER_W
    rsrc = (rsrc0, rsrc1)
    rdst = (rdst0, rdst1)
    rw = (rw0, rw1)
    isem = (isem0, isem1)

    def _issue_blk(k, b):
        base = ebase + k * BLK
        pltpu.async_copy(src_hbm.at[pl.ds(base, BLK)], rsrc[b], isem[b])
        pltpu.async_copy(dst_hbm.at[pl.ds(base, BLK)], rdst[b], isem[b])
        pltpu.async_copy(w_hbm.at[pl.ds(base, BLK)], rw[b], isem[b])

    def _wait_blk(b):
        pltpu.make_async_copy(src_hbm.at[pl.ds(0, BLK)], rsrc[b],
                              isem[b]).wait()
        pltpu.make_async_copy(dst_hbm.at[pl.ds(0, BLK)], rdst[b],
                              isem[b]).wait()
        pltpu.make_async_copy(w_hbm.at[pl.ds(0, BLK)], rw[b],
                              isem[b]).wait()

    _issue_blk(0, 0)
    if NBLK > 1:
        _issue_blk(1, 1)

    cnt0 = jnp.int32(0)
    cnt1 = jnp.int32(0)
    for k in range(NBLK):
        b = k % 2
        _wait_blk(b)

        def _step(i, carry, b=b):
            c0, c1 = carry
            sl = pl.ds(i * L, L)
            sv = rsrc[b][sl]
            dv = rdst[b][sl]
            wv = rw[b][sl]
            m0 = dv < NHALF
            plsc.store_compressed(ls0.at[pl.ds(c0, L)], sv, mask=m0)
            plsc.store_compressed(ld0.at[pl.ds(c0, L)], dv, mask=m0)
            plsc.store_compressed(lw0.at[pl.ds(c0, L)], wv, mask=m0)
            m1 = ~m0
            plsc.store_compressed(ls1.at[pl.ds(c1, L)], sv, mask=m1)
            plsc.store_compressed(ld1.at[pl.ds(c1, L)], dv - NHALF, mask=m1)
            plsc.store_compressed(lw1.at[pl.ds(c1, L)], wv, mask=m1)
            n0 = lax.reduce_max(plsc.all_reduce_population_count(m0), (0,))
            return c0 + n0, c1 + (L - n0)

        cnt0, cnt1 = lax.fori_loop(0, BLK // L, _step, (cnt0, cnt1))
        if k + 2 < NBLK:
            _issue_blk(k + 2, b)

    zero16i = jnp.zeros((L,), jnp.int32)
    zero16f = jnp.zeros((L,), jnp.float32)

    def _pad(cnt, lsb, ldb, lwb):
        npairs = lax.max((cnt + 2 * CHUNK - 1) >> PAIR_SHIFT, 1)
        padded = npairs << PAIR_SHIFT

        def _padv(k, _):
            off = cnt + k * L
            lsb[pl.ds(off, L)] = zero16i
            ldb[pl.ds(off, L)] = zero16i
            lwb[pl.ds(off, L)] = zero16f
            return 0

        lax.fori_loop(0, (padded - cnt + L - 1) >> 4, _padv, 0)
        return npairs

    np0 = _pad(cnt0, ls0, ld0, lw0)
    np1 = _pad(cnt1, ls1, ld1, lw1)

    iota = lax.iota(jnp.int32, L)
    cntv[...] = jnp.where(iota == 0, np0, jnp.where(iota == 1, np1, 0))

    slot0 = (0 * NW + wid) * CAP
    slot1 = (1 * NW + wid) * CAP
    pltpu.async_copy(ls0, srcR.at[pl.ds(slot0, CAP)], osem)
    pltpu.async_copy(ld0, dstR.at[pl.ds(slot0, CAP)], osem)
    pltpu.async_copy(lw0, wR.at[pl.ds(slot0, CAP)], osem)
    pltpu.async_copy(ls1, srcR.at[pl.ds(slot1, CAP)], osem)
    pltpu.async_copy(ld1, dstR.at[pl.ds(slot1, CAP)], osem)
    pltpu.async_copy(lw1, wR.at[pl.ds(slot1, CAP)], osem)
    pltpu.async_copy(cntv, cntR.at[pl.ds(wid * L, L)], osem)
    pltpu.make_async_copy(ls0, srcR.at[pl.ds(slot0, CAP)], osem).wait()
    pltpu.make_async_copy(ld0, dstR.at[pl.ds(slot0, CAP)], osem).wait()
    pltpu.make_async_copy(lw0, wR.at[pl.ds(slot0, CAP)], osem).wait()
    pltpu.make_async_copy(ls1, srcR.at[pl.ds(slot1, CAP)], osem).wait()
    pltpu.make_async_copy(ld1, dstR.at[pl.ds(slot1, CAP)], osem).wait()
    pltpu.make_async_copy(lw1, wR.at[pl.ds(slot1, CAP)], osem).wait()
    pltpu.make_async_copy(cntv, cntR.at[pl.ds(wid * L, L)], osem).wait()


@functools.partial(
    pl.kernel,
    out_type=jax.ShapeDtypeStruct((NC, NHALF, D), jnp.float32),
    mesh=_mesh,
    scratch_types=[
        pltpu.VMEM((2 * CHUNK,), jnp.int32),
        pltpu.VMEM((2 * CHUNK,), jnp.int32),
        pltpu.VMEM((2 * CHUNK,), jnp.int32),
        pltpu.VMEM((2 * CHUNK,), jnp.int32),
        pltpu.VMEM((2 * CHUNK,), jnp.float32),
        pltpu.VMEM((2 * CHUNK,), jnp.float32),
        pltpu.VMEM((CHUNK,), jnp.int32),
        pltpu.VMEM((CHUNK,), jnp.int32),
        pltpu.VMEM((CHUNK,), jnp.int32),
        pltpu.VMEM((CHUNK,), jnp.int32),
        pltpu.VMEM((CHUNK,), jnp.int32),
        pltpu.VMEM((CHUNK,), jnp.int32),
        pltpu.VMEM((CHUNK,), jnp.int32),
        pltpu.VMEM((CHUNK,), jnp.int32),
        pltpu.VMEM((CHUNK,), jnp.float32),
        pltpu.VMEM((CHUNK,), jnp.float32),
        pltpu.VMEM((CHUNK,), jnp.float32),
        pltpu.VMEM((CHUNK,), jnp.float32),
        pltpu.VMEM((CHUNK, D), jnp.float32),
        pltpu.VMEM((CHUNK, D), jnp.float32),
        pltpu.VMEM((CHUNK, D), jnp.float32),
        pltpu.VMEM((CHUNK, D), jnp.float32),
        pltpu.VMEM((L,), jnp.int32),
        pltpu.VMEM((ZROWS, D), jnp.float32),
        pltpu.VMEM((TAIL, D), jnp.float32),
        pltpu.VMEM_SHARED((NHALF, D), jnp.float32),
        pltpu.SemaphoreType.DMA,
        pltpu.SemaphoreType.DMA,
        pltpu.SemaphoreType.DMA,
        pltpu.SemaphoreType.DMA,
        pltpu.SemaphoreType.DMA,
        pltpu.SemaphoreType.DMA,
        pltpu.SemaphoreType.DMA,
        pltpu.SemaphoreType.DMA,
        pltpu.SemaphoreType.DMA,
        pltpu.SemaphoreType.DMA,
    ],
    compiler_params=pltpu.CompilerParams(needs_layout_passes=False),
)
def _propagate_sc(x_hbm, srcR, dstR, wR, cntR, out_hbm,
                  rsrc0, rsrc1, rdst0, rdst1, rw0, rw1,
                  srcv0, srcv1, srcv2, srcv3,
                  dstv0, dstv1, dstv2, dstv3,
                  wv0, wv1, wv2, wv3,
                  rows0, rows1, rows2, rows3, cntb, stage, stail, acc,
                  esem0, esem1, gsem0, gsem1, gsem2, gsem3,
                  ssem0, ssem1, ssem2, ssem3):
    rsrc = (rsrc0, rsrc1)
    rdst = (rdst0, rdst1)
    rw = (rw0, rw1)
    srcv = (srcv0, srcv1, srcv2, srcv3)
    dstv = (dstv0, dstv1, dstv2, dstv3)
    wv = (wv0, wv1, wv2, wv3)
    rows = (rows0, rows1, rows2, rows3)
    esem = (esem0, esem1)
    gsem = (gsem0, gsem1, gsem2, gsem3)
    ssem = (ssem0, ssem1, ssem2, ssem3)
    c = lax.axis_index("c")
    s = lax.axis_index("s")

    zero = jnp.zeros((L,), jnp.float32)

    def _zrow(i, _):
        for j in range(D // L):
            stage[i, pl.ds(j * L, L)] = zero
        return 0

    lax.fori_loop(0, ZROWS, _zrow, 0)
    for i in range(TAIL):
        for j in range(D // L):
            stail[i, pl.ds(j * L, L)] = zero
    pltpu.sync_copy(stage, acc.at[pl.ds(s * ZROWS, ZROWS)])

    @pl.when(s == NS - 1)
    def _():
        pltpu.sync_copy(stail, acc.at[pl.ds(NS * ZROWS, TAIL)])

    plsc.subcore_barrier()

    iota = lax.iota(jnp.int32, L)

    def _issue_e(base0, j, b):
        base = base0 + j * 2 * CHUNK
        pltpu.async_copy(srcR.at[pl.ds(base, 2 * CHUNK)], rsrc[b], esem[b])
        pltpu.async_copy(dstR.at[pl.ds(base, 2 * CHUNK)], rdst[b], esem[b])
        pltpu.async_copy(wR.at[pl.ds(base, 2 * CHUNK)], rw[b], esem[b])

    def _wait_e(b):
        pltpu.make_async_copy(srcR.at[pl.ds(0, 2 * CHUNK)], rsrc[b],
                              esem[b]).wait()
        pltpu.make_async_copy(dstR.at[pl.ds(0, 2 * CHUNK)], rdst[b],
                              esem[b]).wait()
        pltpu.make_async_copy(wR.at[pl.ds(0, 2 * CHUNK)], rw[b],
                              esem[b]).wait()

    def _build(eb, half, b):
        for k in range(CHUNK // L):
            sl = pl.ds(k * L, L)
            el = pl.ds(half * CHUNK + k * L, L)
            srcv[b][sl] = rsrc[eb][el]
            dstv[b][sl] = rdst[eb][el]
            wv[b][sl] = rw[eb][el]

    def _issue_g(b):
        h = CHUNK // 2
        pltpu.async_copy(x_hbm.at[srcv[b].at[pl.ds(0, h)]],
                         rows[b].at[pl.ds(0, h)], gsem[b])
        pltpu.async_copy(x_hbm.at[srcv[b].at[pl.ds(h, h)]],
                         rows[b].at[pl.ds(h, h)], gsem[b])

    def _wait_g(b):
        pltpu.make_async_copy(x_hbm.at[pl.ds(0, CHUNK)], rows[b],
                              gsem[b]).wait()

    def _scale(b):
        @plsc.parallel_loop(0, CHUNK, unroll=2)
        def _edge(e):
            wb = plsc.load_gather(wv[b], [jnp.full((L,), e, jnp.int32)])
            for j in range(D // L):
                sl = pl.ds(j * L, L)
                rows[b][e, sl] = rows[b][e, sl] * wb

    def _issue_a(b):
        pltpu.async_copy(rows[b], acc.at[dstv[b]], ssem[b], add=True)

    def _wait_a(b):
        pltpu.make_async_copy(x_hbm.at[pl.ds(0, CHUNK)], rows[b],
                              ssem[b]).wait()

    def _run_slot(w):
        pltpu.sync_copy(cntR.at[pl.ds(w * L, L)], cntb)
        npair = lax.reduce_max(jnp.where(iota == c, cntb[...], 0), (0,))
        base0 = (c * NW + w) * CAP

        nch = npair * 2

        _issue_e(base0, 0, 0)
        _issue_e(base0, 1, 1)
        _wait_e(0)
        _build(0, 0, 0)
        _issue_g(0)
        _build(0, 1, 1)
        _issue_g(1)

        @pl.when(2 < npair)
        def _():
            _issue_e(base0, 2, 0)

        def _quad(i, _):
            for k in range(4):
                ck = 4 * i + k
                p = k
                np_ = (k + 2) % 4
                ebb = 1 if k < 2 else 0
                half = k % 2

                @pl.when(ck < nch)
                def _(ck=ck, p=p, np_=np_, ebb=ebb, half=half, k=k):
                    @pl.when(ck + 2 < nch)
                    def _():
                        if k % 2 == 0:
                            _wait_e(ebb)
                        if k >= 2:
                            _wait_a(np_)
                        else:
                            @pl.when(ck >= 2)
                            def _():
                                _wait_a(np_)
                        _build(ebb, half, np_)
                        _issue_g(np_)
                        if k % 2 == 1:
                            jj = (ck + 2) // 2 + 2

                            @pl.when(jj < npair)
                            def _():
                                _issue_e(base0, jj, ebb)

                    _wait_g(p)
                    _scale(p)
                    _issue_a(p)
            return 0

        lax.fori_loop(0, (nch + 3) >> 2, _quad, 0)

        @pl.when(npair == 1)
        def _():
            _wait_e(1)
        _wait_a(0)
        _wait_a(1)

        @pl.when(npair >= 2)
        def _():
            _wait_a(2)
            _wait_a(3)

    _run_slot(s * 2)
    _run_slot(s * 2 + 1)
    plsc.subcore_barrier()

    r0 = s * ZROWS
    pltpu.sync_copy(acc.at[pl.ds(r0, ZROWS)], stage)
    pltpu.sync_copy(stage, out_hbm.at[c, pl.ds(r0, ZROWS)])

    @pl.when(s == NS - 1)
    def _():
        pltpu.sync_copy(acc.at[pl.ds(NS * ZROWS, TAIL)], stail)
        pltpu.sync_copy(stail, out_hbm.at[c, pl.ds(NS * ZROWS, TAIL)])


def _dense_relu_body(p_ref, wt_ref, b_ref, out_ref):
    y = jnp.dot(p_ref[...], wt_ref[...],
                preferred_element_type=jnp.float32) + b_ref[...]
    out_ref[...] = jnp.maximum(y, 0.0)


def _dense_final_body(p_ref, wt_ref, b_ref, out_ref):
    y = jnp.dot(p_ref[...], wt_ref[...],
                preferred_element_type=jnp.float32) + b_ref[...]
    m = jnp.max(y, axis=1, keepdims=True)
    e = y - m
    lse = jnp.log(jnp.sum(jnp.exp(e), axis=1, keepdims=True))
    out_ref[...] = e - lse


_ROW_BLK = 2000


def _dense_call(body, p, wt, b):
    grid = (N_NODES // _ROW_BLK,)
    return pl.pallas_call(
        body,
        grid=grid,
        in_specs=[
            pl.BlockSpec((_ROW_BLK, D), lambda i: (i, 0)),
            pl.BlockSpec((D, D), lambda i: (0, 0)),
            pl.BlockSpec((1, D), lambda i: (0, 0)),
        ],
        out_specs=pl.BlockSpec((_ROW_BLK, D), lambda i: (i, 0)),
        out_shape=jax.ShapeDtypeStruct((N_NODES, D), jnp.float32),
    )(p, wt, b)


def kernel(x, edge_index, edge_weight, W1, b1, W2, b2, W3, b3):
    src = edge_index[0].astype(jnp.int32)
    dst = edge_index[1].astype(jnp.int32)
    w = edge_weight.astype(jnp.float32)

    srcR, dstR, wR, cntR = _route_sc(src, dst, w)

    p1 = _propagate_sc(x, srcR, dstR, wR, cntR).reshape(N_NODES, D)
    h1 = _dense_call(_dense_relu_body, p1, W1.T, b1.reshape(1, D))
    p2 = _propagate_sc(h1, srcR, dstR, wR, cntR).reshape(N_NODES, D)
    h2 = _dense_call(_dense_relu_body, p2, W2.T, b2.reshape(1, D))
    p3 = _propagate_sc(h2, srcR, dstR, wR, cntR).reshape(N_NODES, D)
    return _dense_call(_dense_final_body, p3, W3.T, b3.reshape(1, D))
